# Initial kernel scaffold; baseline (speedup 1.0000x reference)
import jax
import jax.numpy as jnp
from jax.experimental import pallas as pl


def kernel(toks, unberter, tokborders, wordborders, rels, emb, Wb, Wr):
    raise NotImplementedError("write your pallas kernel here")


# speedup vs baseline: 1.7950x; 1.7950x over previous
import functools

import jax
import jax.numpy as jnp
from jax import lax
from jax.experimental import pallas as pl
from jax.experimental.pallas import tpu as pltpu
from jax.experimental.pallas import tpu_sc as plsc

B, S, D, V, R = 64, 512, 256, 30000, 200

_VBLK = 1024
_VPAD = ((V + _VBLK - 1) // _VBLK) * _VBLK


def _p_body(emb_ref, wb_ref, p0_ref, p1_ref):
    e = emb_ref[...]
    wb = wb_ref[...]
    p0_ref[...] = jnp.sum(e * wb[:, 0][None, :], axis=1)
    p1_ref[...] = jnp.sum(e * wb[:, 1][None, :], axis=1)


def _p_call(emb, Wb):
    return pl.pallas_call(
        _p_body,
        grid=(_VPAD // _VBLK,),
        in_specs=[
            pl.BlockSpec((_VBLK, D), lambda i: (i, 0)),
            pl.BlockSpec((D, 2), lambda i: (0, 0)),
        ],
        out_specs=[
            pl.BlockSpec((_VBLK,), lambda i: (i,)),
            pl.BlockSpec((_VBLK,), lambda i: (i,)),
        ],
        out_shape=[
            jax.ShapeDtypeStruct((_VPAD,), jnp.float32),
            jax.ShapeDtypeStruct((_VPAD,), jnp.float32),
        ],
    )(emb, Wb)


_NW = 32
_BPW = B // _NW
_CH = 128
_NCH = S // _CH


def _sc_body(toks_hbm, emb_hbm, p0_hbm, p1_hbm,
             hsum_hbm, bp0_hbm, bp1_hbm,
             toks_v, chunk_v, p0_v, p1_v, bp0_v, bp1_v, hsum_v, sem):
    c = lax.axis_index("c")
    s = lax.axis_index("s")
    wid = c * 16 + s
    base = wid * _BPW

    pltpu.sync_copy(toks_hbm.at[pl.ds(base, _BPW)], toks_v)
    pltpu.sync_copy(p0_hbm, p0_v)
    pltpu.sync_copy(p1_hbm, p1_v)

    for local in range(_BPW):
        def bp_body(i, carry):
            idx = toks_v[local, pl.ds(i * 16, 16)]
            bp0_v[local, pl.ds(i * 16, 16)] = plsc.load_gather(p0_v, [idx])
            bp1_v[local, pl.ds(i * 16, 16)] = plsc.load_gather(p1_v, [idx])
            return carry
        lax.fori_loop(0, S // 16, bp_body, 0)

        acc = tuple(jnp.zeros((16,), jnp.float32) for _ in range(D // 16))
        for ck in range(_NCH):
            pltpu.async_copy(
                emb_hbm.at[toks_v.at[local, pl.ds(ck * _CH, _CH)]],
                chunk_v, sem).wait()

            def row_body(r, a):
                return tuple(a[d] + chunk_v[r, pl.ds(d * 16, 16)]
                             for d in range(D // 16))
            acc = lax.fori_loop(0, _CH, row_body, acc)
        for d in range(D // 16):
            hsum_v[local, pl.ds(d * 16, 16)] = acc[d]

    pltpu.sync_copy(hsum_v, hsum_hbm.at[pl.ds(base, _BPW)])
    pltpu.sync_copy(bp0_v, bp0_hbm.at[pl.ds(base, _BPW)])
    pltpu.sync_copy(bp1_v, bp1_hbm.at[pl.ds(base, _BPW)])


def _sc_call(toks, emb, p0, p1):
    mesh = plsc.VectorSubcoreMesh(core_axis_name="c", subcore_axis_name="s")
    f = pl.kernel(
        _sc_body,
        out_type=(
            jax.ShapeDtypeStruct((B, D), jnp.float32),
            jax.ShapeDtypeStruct((B, S), jnp.float32),
            jax.ShapeDtypeStruct((B, S), jnp.float32),
        ),
        mesh=mesh,
        compiler_params=pltpu.CompilerParams(needs_layout_passes=False),
        scratch_types=[
            pltpu.VMEM((_BPW, S), jnp.int32),
            pltpu.VMEM((_CH, D), jnp.float32),
            pltpu.VMEM((_VPAD,), jnp.float32),
            pltpu.VMEM((_VPAD,), jnp.float32),
            pltpu.VMEM((_BPW, S), jnp.float32),
            pltpu.VMEM((_BPW, S), jnp.float32),
            pltpu.VMEM((_BPW, D), jnp.float32),
            pltpu.SemaphoreType.DMA,
        ],
    )
    return f(toks, emb, p0, p1)


_BB = 8


def _loss_body(bp0_ref, bp1_ref, u_ref, hsum_ref, tb_ref, wb_ref, rels_ref, wr_ref,
               allces_ref, bces_ref, rces_ref, f1_ref, bacc_ref, racc_ref, both_ref):
    bp0 = bp0_ref[...]
    bp1 = bp1_ref[...]
    u = u_ref[...]
    tb = tb_ref[...]
    wb = wb_ref[...]
    rels = rels_ref[...][:, 0]

    iota_s = lax.broadcasted_iota(jnp.int32, (_BB, S), 1)

    def ce_and_probs(bp, gold):
        m = jnp.max(bp, axis=1, keepdims=True)
        e = jnp.exp(bp - m)
        se = jnp.sum(e, axis=1, keepdims=True)
        lse = jnp.log(se)[:, 0] + m[:, 0]
        gl = jnp.sum(jnp.where(iota_s == gold[:, None], bp, 0.0), axis=1)
        return lse - gl, e / se

    ce0, probs0 = ce_and_probs(bp0, tb[:, 0])
    ce1, probs1 = ce_and_probs(bp1, tb[:, 1])
    bces = 0.5 * (ce0 + ce1)

    jcols = lax.broadcasted_iota(jnp.int32, (S, S), 1) + 2
    iota_2s = lax.broadcasted_iota(jnp.int32, (2, S), 1)
    ams = []
    for i in range(_BB):
        oh = (u[i][:, None] == jcols).astype(jnp.float32)
        probs_i = jnp.concatenate([probs0[i][None, :], probs1[i][None, :]], axis=0)
        wbp = lax.dot(probs_i, oh, precision=lax.Precision.HIGHEST,
                      preferred_element_type=jnp.float32)
        mx = jnp.max(wbp, axis=1, keepdims=True)
        am = jnp.min(jnp.where(wbp == mx, iota_2s, S), axis=1)
        ams.append(am[None, :])
    am_all = jnp.concatenate(ams, axis=0)

    bacc = jnp.all(am_all == wb, axis=1).astype(jnp.float32)
    ps = am_all[:, 0].astype(jnp.float32)
    pe = am_all[:, 1].astype(jnp.float32)
    gs = wb[:, 0].astype(jnp.float32)
    ge = wb[:, 1].astype(jnp.float32)
    ovl = jnp.maximum(jnp.minimum(pe, ge) - jnp.maximum(ps, gs), 0.0)
    rec = ovl / jnp.maximum(ge - gs, 1e-6)
    prc = ovl / jnp.maximum(pe - ps, 1e-6)
    f1 = 2.0 * rec * prc / jnp.maximum(rec + prc, 1e-6)

    hs = hsum_ref[...] * jnp.float32(1.0 / S)
    rp = lax.dot(hs, wr_ref[...], precision=lax.Precision.HIGHEST,
                 preferred_element_type=jnp.float32)
    iota_r = lax.broadcasted_iota(jnp.int32, (_BB, R), 1)
    mr = jnp.max(rp, axis=1, keepdims=True)
    er = jnp.exp(rp - mr)
    ser = jnp.sum(er, axis=1, keepdims=True)
    lser = jnp.log(ser)[:, 0] + mr[:, 0]
    glr = jnp.sum(jnp.where(iota_r == rels[:, None], rp, 0.0), axis=1)
    rces = lser - glr
    am_r = jnp.min(jnp.where(rp == mr, iota_r, R), axis=1)
    racc = (am_r == rels).astype(jnp.float32)

    allces_ref[...] = (bces + rces)[:, None]
    bces_ref[...] = bces[:, None]
    rces_ref[...] = rces[:, None]
    f1_ref[...] = f1[:, None]
    bacc_ref[...] = bacc[:, None]
    racc_ref[...] = racc[:, None]
    both_ref[...] = (bacc * racc)[:, None]


def _loss_call(bp0, bp1, u, hsum, tb, wb, rels, Wr):
    vec = lambda: pl.BlockSpec((_BB, 1), lambda i: (i, 0))
    return pl.pallas_call(
        _loss_body,
        grid=(B // _BB,),
        in_specs=[
            pl.BlockSpec((_BB, S), lambda i: (i, 0)),
            pl.BlockSpec((_BB, S), lambda i: (i, 0)),
            pl.BlockSpec((_BB, S), lambda i: (i, 0)),
            pl.BlockSpec((_BB, D), lambda i: (i, 0)),
            pl.BlockSpec((_BB, 2), lambda i: (i, 0)),
            pl.BlockSpec((_BB, 2), lambda i: (i, 0)),
            vec(),
            pl.BlockSpec((D, R), lambda i: (0, 0)),
        ],
        out_specs=[vec()] * 7,
        out_shape=[jax.ShapeDtypeStruct((B, 1), jnp.float32)] * 7,
    )(bp0, bp1, u, hsum, tb, wb, rels.reshape(B, 1), Wr)


def kernel(toks, unberter, tokborders, wordborders, rels, emb, Wb, Wr):
    toks = toks.astype(jnp.int32)
    unberter = unberter.astype(jnp.int32)
    tokborders = tokborders.astype(jnp.int32)
    wordborders = wordborders.astype(jnp.int32)
    rels = rels.astype(jnp.int32)
    emb = emb.astype(jnp.float32)
    Wb = Wb.astype(jnp.float32)
    Wr = Wr.astype(jnp.float32)

    p0, p1 = _p_call(emb, Wb)
    hsum, bp0, bp1 = _sc_call(toks, emb, p0, p1)
    allces, bces, rces, f1, bacc, racc, both = _loss_call(
        bp0, bp1, unberter, hsum, tokborders, wordborders, rels, Wr)
    flat = lambda x: x.reshape(B)
    return (flat(allces), flat(bces), flat(rces), f1,
            flat(bacc), flat(racc), flat(both))


# speedup vs baseline: 2.3910x; 1.3320x over previous
import functools

import jax
import jax.numpy as jnp
from jax import lax
from jax.experimental import pallas as pl
from jax.experimental.pallas import tpu as pltpu
from jax.experimental.pallas import tpu_sc as plsc

B, S, D, V, R = 64, 512, 256, 30000, 200

_VBLK = 1024
_VPAD = ((V + _VBLK - 1) // _VBLK) * _VBLK


def _p_body(emb_ref, wb_ref, p0_ref, p1_ref):
    e = emb_ref[...]
    wb = wb_ref[...]
    p0_ref[...] = jnp.sum(e * wb[:, 0][None, :], axis=1)
    p1_ref[...] = jnp.sum(e * wb[:, 1][None, :], axis=1)


def _p_call(emb, Wb):
    return pl.pallas_call(
        _p_body,
        grid=(_VPAD // _VBLK,),
        in_specs=[
            pl.BlockSpec((_VBLK, D), lambda i: (i, 0)),
            pl.BlockSpec((D, 2), lambda i: (0, 0)),
        ],
        out_specs=[
            pl.BlockSpec((_VBLK,), lambda i: (i,)),
            pl.BlockSpec((_VBLK,), lambda i: (i,)),
        ],
        out_shape=[
            jax.ShapeDtypeStruct((_VPAD,), jnp.float32),
            jax.ShapeDtypeStruct((_VPAD,), jnp.float32),
        ],
    )(emb, Wb)


_NW = 32
_BPW = B // _NW
_CH = 128
_NCH = S // _CH


def _sc_body(toks_hbm, emb_hbm, p0_hbm, p1_hbm,
             hsum_hbm, bp0_hbm, bp1_hbm,
             toks_v, chunk_v, p0_v, p1_v, bp0_v, bp1_v, hsum_v, sem):
    c = lax.axis_index("c")
    s = lax.axis_index("s")
    wid = c * 16 + s
    base = wid * _BPW

    pltpu.sync_copy(toks_hbm.at[pl.ds(base, _BPW)], toks_v)
    pltpu.sync_copy(p0_hbm, p0_v)
    pltpu.sync_copy(p1_hbm, p1_v)

    for local in range(_BPW):
        def bp_body(i, carry):
            idx = toks_v[local, pl.ds(i * 16, 16)]
            bp0_v[local, pl.ds(i * 16, 16)] = plsc.load_gather(p0_v, [idx])
            bp1_v[local, pl.ds(i * 16, 16)] = plsc.load_gather(p1_v, [idx])
            return carry
        lax.fori_loop(0, S // 16, bp_body, 0)

        acc = tuple(jnp.zeros((16,), jnp.float32) for _ in range(D // 16))
        for ck in range(_NCH):
            pltpu.async_copy(
                emb_hbm.at[toks_v.at[local, pl.ds(ck * _CH, _CH)]],
                chunk_v, sem).wait()

            def row_body(r, a):
                return tuple(a[d] + chunk_v[r, pl.ds(d * 16, 16)]
                             for d in range(D // 16))
            acc = lax.fori_loop(0, _CH, row_body, acc)
        for d in range(D // 16):
            hsum_v[local, pl.ds(d * 16, 16)] = acc[d]

    pltpu.sync_copy(hsum_v, hsum_hbm.at[pl.ds(base, _BPW)])
    pltpu.sync_copy(bp0_v, bp0_hbm.at[pl.ds(base, _BPW)])
    pltpu.sync_copy(bp1_v, bp1_hbm.at[pl.ds(base, _BPW)])


def _sc_call(toks, emb, p0, p1):
    mesh = plsc.VectorSubcoreMesh(core_axis_name="c", subcore_axis_name="s")
    f = pl.kernel(
        _sc_body,
        out_type=(
            jax.ShapeDtypeStruct((B, D), jnp.float32),
            jax.ShapeDtypeStruct((B, S), jnp.float32),
            jax.ShapeDtypeStruct((B, S), jnp.float32),
        ),
        mesh=mesh,
        compiler_params=pltpu.CompilerParams(needs_layout_passes=False),
        scratch_types=[
            pltpu.VMEM((_BPW, S), jnp.int32),
            pltpu.VMEM((_CH, D), jnp.float32),
            pltpu.VMEM((_VPAD,), jnp.float32),
            pltpu.VMEM((_VPAD,), jnp.float32),
            pltpu.VMEM((_BPW, S), jnp.float32),
            pltpu.VMEM((_BPW, S), jnp.float32),
            pltpu.VMEM((_BPW, D), jnp.float32),
            pltpu.SemaphoreType.DMA,
        ],
    )
    return f(toks, emb, p0, p1)


_BB = B


def _loss_body(bp0_ref, bp1_ref, u_ref, hsum_ref, tb_ref, wb_ref, rels_ref, wr_ref,
               allces_ref, bces_ref, rces_ref, f1_ref, bacc_ref, racc_ref, both_ref):
    bp0 = bp0_ref[...]
    bp1 = bp1_ref[...]
    u = u_ref[...]
    tb = tb_ref[...]
    wb = wb_ref[...]
    rels = rels_ref[...][:, 0]

    iota_s = lax.broadcasted_iota(jnp.int32, (_BB, S), 1)

    def ce_and_probs(bp, gold):
        m = jnp.max(bp, axis=1, keepdims=True)
        e = jnp.exp(bp - m)
        se = jnp.sum(e, axis=1, keepdims=True)
        lse = jnp.log(se)[:, 0] + m[:, 0]
        gl = jnp.sum(jnp.where(iota_s == gold[:, None], bp, 0.0), axis=1)
        return lse - gl, e

    ce0, probs0 = ce_and_probs(bp0, tb[:, 0])
    ce1, probs1 = ce_and_probs(bp1, tb[:, 1])
    bces = 0.5 * (ce0 + ce1)

    def cum_op(x, op):
        sh = 1
        while sh < S:
            shifted = jnp.concatenate(
                [jnp.zeros((_BB, sh), jnp.float32), x[:, :-sh]], axis=1)
            x = op(x, shifted)
            sh *= 2
        return x

    nxt = jnp.concatenate([u[:, 1:], jnp.full((_BB, 1), S + 2, jnp.int32)],
                          axis=1)
    bmask = u != nxt
    valid = bmask & (u >= 2)

    def seg_argmax(e):
        c = cum_op(e, lax.add)
        cb = jnp.where(bmask, c, 0.0)
        cm = cum_op(cb, lax.max)
        prev = jnp.concatenate(
            [jnp.zeros((_BB, 1), jnp.float32), cm[:, :-1]], axis=1)
        seg = c - prev
        segv = jnp.where(valid, seg, -1.0)
        mx = jnp.max(segv, axis=1, keepdims=True)
        jpick = jnp.min(jnp.where(segv == mx, u - 2, jnp.int32(1 << 20)),
                        axis=1)
        return jnp.where(mx[:, 0] > 0.0, jpick, 0)

    am_all = jnp.concatenate(
        [seg_argmax(probs0)[:, None], seg_argmax(probs1)[:, None]], axis=1)

    bacc = jnp.all(am_all == wb, axis=1).astype(jnp.float32)
    ps = am_all[:, 0].astype(jnp.float32)
    pe = am_all[:, 1].astype(jnp.float32)
    gs = wb[:, 0].astype(jnp.float32)
    ge = wb[:, 1].astype(jnp.float32)
    ovl = jnp.maximum(jnp.minimum(pe, ge) - jnp.maximum(ps, gs), 0.0)
    rec = ovl / jnp.maximum(ge - gs, 1e-6)
    prc = ovl / jnp.maximum(pe - ps, 1e-6)
    f1 = 2.0 * rec * prc / jnp.maximum(rec + prc, 1e-6)

    hs = hsum_ref[...] * jnp.float32(1.0 / S)
    rp = lax.dot(hs, wr_ref[...], precision=lax.Precision.HIGHEST,
                 preferred_element_type=jnp.float32)
    iota_r = lax.broadcasted_iota(jnp.int32, (_BB, R), 1)
    mr = jnp.max(rp, axis=1, keepdims=True)
    er = jnp.exp(rp - mr)
    ser = jnp.sum(er, axis=1, keepdims=True)
    lser = jnp.log(ser)[:, 0] + mr[:, 0]
    glr = jnp.sum(jnp.where(iota_r == rels[:, None], rp, 0.0), axis=1)
    rces = lser - glr
    am_r = jnp.min(jnp.where(rp == mr, iota_r, R), axis=1)
    racc = (am_r == rels).astype(jnp.float32)

    allces_ref[...] = (bces + rces)[:, None]
    bces_ref[...] = bces[:, None]
    rces_ref[...] = rces[:, None]
    f1_ref[...] = f1[:, None]
    bacc_ref[...] = bacc[:, None]
    racc_ref[...] = racc[:, None]
    both_ref[...] = (bacc * racc)[:, None]


def _loss_call(bp0, bp1, u, hsum, tb, wb, rels, Wr):
    return pl.pallas_call(
        _loss_body,
        out_shape=[jax.ShapeDtypeStruct((B, 1), jnp.float32)] * 7,
    )(bp0, bp1, u, hsum, tb, wb, rels.reshape(B, 1), Wr)


def kernel(toks, unberter, tokborders, wordborders, rels, emb, Wb, Wr):
    toks = toks.astype(jnp.int32)
    unberter = unberter.astype(jnp.int32)
    tokborders = tokborders.astype(jnp.int32)
    wordborders = wordborders.astype(jnp.int32)
    rels = rels.astype(jnp.int32)
    emb = emb.astype(jnp.float32)
    Wb = Wb.astype(jnp.float32)
    Wr = Wr.astype(jnp.float32)

    p0, p1 = _p_call(emb, Wb)
    hsum, bp0, bp1 = _sc_call(toks, emb, p0, p1)
    allces, bces, rces, f1, bacc, racc, both = _loss_call(
        bp0, bp1, unberter, hsum, tokborders, wordborders, rels, Wr)
    flat = lambda x: x.reshape(B)
    return (flat(allces), flat(bces), flat(rces), f1,
            flat(bacc), flat(racc), flat(both))


# speedup vs baseline: 2.8640x; 1.1978x over previous
import functools

import jax
import jax.numpy as jnp
from jax import lax
from jax.experimental import pallas as pl
from jax.experimental.pallas import tpu as pltpu
from jax.experimental.pallas import tpu_sc as plsc

B, S, D, V, R = 64, 512, 256, 30000, 200

_VBLK = 2048
_VPAD = ((V + _VBLK - 1) // _VBLK) * _VBLK


def _p_body(emb_ref, wb_ref, p0_ref, p1_ref):
    pd = lax.dot(emb_ref[...], wb_ref[...],
                 precision=lax.Precision.HIGHEST,
                 preferred_element_type=jnp.float32)
    p0_ref[...] = pd[:, 0]
    p1_ref[...] = pd[:, 1]


def _p_call(emb, Wb):
    return pl.pallas_call(
        _p_body,
        grid=(_VPAD // _VBLK,),
        in_specs=[
            pl.BlockSpec((_VBLK, D), lambda i: (i, 0)),
            pl.BlockSpec((D, 2), lambda i: (0, 0)),
        ],
        out_specs=[
            pl.BlockSpec((_VBLK,), lambda i: (i,)),
            pl.BlockSpec((_VBLK,), lambda i: (i,)),
        ],
        out_shape=[
            jax.ShapeDtypeStruct((_VPAD,), jnp.float32),
            jax.ShapeDtypeStruct((_VPAD,), jnp.float32),
        ],
    )(emb, Wb)


_NW = 32
_BPW = B // _NW
_CH = 64
_NCH = S // _CH


def _sc_body(toks_hbm, emb_hbm, p0_hbm, p1_hbm, hsum_hbm, bp0_hbm, bp1_hbm,
             toks_v, buf0, buf1, p0_v, p1_v, bp0_v, bp1_v, hsum_v,
             sem0, sem1, semp):
    c = lax.axis_index("c")
    s = lax.axis_index("s")
    base = (c * 16 + s) * _BPW

    pdesc0 = pltpu.async_copy(p0_hbm, p0_v, semp)
    pdesc1 = pltpu.async_copy(p1_hbm, p1_v, semp)
    pltpu.sync_copy(toks_hbm.at[pl.ds(base, _BPW)], toks_v)

    bufs = (buf0, buf1)
    sems = (sem0, sem1)
    ng = _BPW * _NCH
    descs = [None, None]
    acc = None
    for g in range(ng + 1):
        if g < ng:
            local, ck = divmod(g, _NCH)
            descs[g % 2] = pltpu.async_copy(
                emb_hbm.at[toks_v.at[local, pl.ds(ck * _CH, _CH)]],
                bufs[g % 2], sems[g % 2])
        if g >= 1:
            p = g - 1
            plocal, pck = divmod(p, _NCH)
            if pck == 0:
                acc = tuple(jnp.zeros((16,), jnp.float32)
                            for _ in range(D // 16))
            descs[p % 2].wait()
            buf = bufs[p % 2]

            def row_body(r, a, _buf=buf):
                return tuple(a[d] + _buf[r, pl.ds(d * 16, 16)]
                             for d in range(D // 16))
            acc = lax.fori_loop(0, _CH, row_body, acc)
            if pck == _NCH - 1:
                for d in range(D // 16):
                    hsum_v[plocal, pl.ds(d * 16, 16)] = acc[d]

    pltpu.sync_copy(hsum_v, hsum_hbm.at[pl.ds(base, _BPW)])

    pdesc0.wait()
    pdesc1.wait()
    for local in range(_BPW):
        def ex_body(i, carry):
            idx = toks_v[local, pl.ds(i * 16, 16)]
            bp0_v[local, pl.ds(i * 16, 16)] = plsc.load_gather(p0_v, [idx])
            bp1_v[local, pl.ds(i * 16, 16)] = plsc.load_gather(p1_v, [idx])
            return carry
        lax.fori_loop(0, S // 16, ex_body, 0)

    pltpu.sync_copy(bp0_v, bp0_hbm.at[pl.ds(base, _BPW)])
    pltpu.sync_copy(bp1_v, bp1_hbm.at[pl.ds(base, _BPW)])


def _sc_call(toks, emb, p0, p1):
    mesh = plsc.VectorSubcoreMesh(core_axis_name="c", subcore_axis_name="s")
    f = pl.kernel(
        _sc_body,
        out_type=(
            jax.ShapeDtypeStruct((B, D), jnp.float32),
            jax.ShapeDtypeStruct((B, S), jnp.float32),
            jax.ShapeDtypeStruct((B, S), jnp.float32),
        ),
        mesh=mesh,
        compiler_params=pltpu.CompilerParams(needs_layout_passes=False),
        scratch_types=[
            pltpu.VMEM((_BPW, S), jnp.int32),
            pltpu.VMEM((_CH, D), jnp.float32),
            pltpu.VMEM((_CH, D), jnp.float32),
            pltpu.VMEM((_VPAD,), jnp.float32),
            pltpu.VMEM((_VPAD,), jnp.float32),
            pltpu.VMEM((_BPW, S), jnp.float32),
            pltpu.VMEM((_BPW, S), jnp.float32),
            pltpu.VMEM((_BPW, D), jnp.float32),
            pltpu.SemaphoreType.DMA,
            pltpu.SemaphoreType.DMA,
            pltpu.SemaphoreType.DMA,
        ],
    )
    return f(toks, emb, p0, p1)


_BB = B


def _loss_body(bp0_ref, bp1_ref, u_ref, hsum_ref, tb_ref, wb_ref, rels_ref, wr_ref,
               allces_ref, bces_ref, rces_ref, f1_ref, bacc_ref, racc_ref, both_ref):
    bp0 = bp0_ref[...]
    bp1 = bp1_ref[...]
    u = u_ref[...]
    tb = tb_ref[...]
    wb = wb_ref[...]
    rels = rels_ref[...]

    iota_s = lax.broadcasted_iota(jnp.int32, (_BB, S), 1)

    def ce_and_probs(bp, gold):
        m = jnp.max(bp, axis=1, keepdims=True)
        e = jnp.exp(bp - m)
        se = jnp.sum(e, axis=1, keepdims=True)
        lse = jnp.log(se)[:, 0] + m[:, 0]
        gl = jnp.sum(jnp.where(iota_s == gold[:, None], bp, 0.0), axis=1)
        return lse - gl, e

    ce0, probs0 = ce_and_probs(bp0, tb[:, 0])
    ce1, probs1 = ce_and_probs(bp1, tb[:, 1])
    bces = 0.5 * (ce0 + ce1)

    def cum_op(x, op):
        sh = 1
        while sh < S:
            shifted = jnp.concatenate(
                [jnp.zeros((_BB, sh), jnp.float32), x[:, :-sh]], axis=1)
            x = op(x, shifted)
            sh *= 2
        return x

    nxt = jnp.concatenate([u[:, 1:], jnp.full((_BB, 1), S + 2, jnp.int32)],
                          axis=1)
    bmask = u != nxt
    valid = bmask & (u >= 2)

    def seg_argmax(e):
        c = cum_op(e, lax.add)
        cb = jnp.where(bmask, c, 0.0)
        cm = cum_op(cb, lax.max)
        prev = jnp.concatenate(
            [jnp.zeros((_BB, 1), jnp.float32), cm[:, :-1]], axis=1)
        seg = c - prev
        segv = jnp.where(valid, seg, -1.0)
        mx = jnp.max(segv, axis=1, keepdims=True)
        jpick = jnp.min(jnp.where(segv == mx, u - 2, jnp.int32(1 << 20)),
                        axis=1)
        return jnp.where(mx[:, 0] > 0.0, jpick, 0)

    am_all = jnp.concatenate(
        [seg_argmax(probs0)[:, None], seg_argmax(probs1)[:, None]], axis=1)

    bacc = jnp.all(am_all == wb, axis=1).astype(jnp.float32)
    ps = am_all[:, 0].astype(jnp.float32)
    pe = am_all[:, 1].astype(jnp.float32)
    gs = wb[:, 0].astype(jnp.float32)
    ge = wb[:, 1].astype(jnp.float32)
    ovl = jnp.maximum(jnp.minimum(pe, ge) - jnp.maximum(ps, gs), 0.0)
    rec = ovl / jnp.maximum(ge - gs, 1e-6)
    prc = ovl / jnp.maximum(pe - ps, 1e-6)
    f1 = 2.0 * rec * prc / jnp.maximum(rec + prc, 1e-6)

    hs = hsum_ref[...] * jnp.float32(1.0 / S)
    rp = lax.dot(hs, wr_ref[...], precision=lax.Precision.HIGHEST,
                 preferred_element_type=jnp.float32)
    iota_r = lax.broadcasted_iota(jnp.int32, (_BB, R), 1)
    mr = jnp.max(rp, axis=1, keepdims=True)
    er = jnp.exp(rp - mr)
    ser = jnp.sum(er, axis=1, keepdims=True)
    lser = jnp.log(ser)[:, 0] + mr[:, 0]
    glr = jnp.sum(jnp.where(iota_r == rels[:, None], rp, 0.0), axis=1)
    rces = lser - glr
    am_r = jnp.min(jnp.where(rp == mr, iota_r, R), axis=1)
    racc = (am_r == rels).astype(jnp.float32)

    allces_ref[...] = bces + rces
    bces_ref[...] = bces
    rces_ref[...] = rces
    f1_ref[...] = f1[:, None]
    bacc_ref[...] = bacc
    racc_ref[...] = racc
    both_ref[...] = bacc * racc


def _loss_call(bp0, bp1, u, hsum, tb, wb, rels, Wr):
    vec = jax.ShapeDtypeStruct((B,), jnp.float32)
    return pl.pallas_call(
        _loss_body,
        out_shape=[vec, vec, vec,
                   jax.ShapeDtypeStruct((B, 1), jnp.float32),
                   vec, vec, vec],
    )(bp0, bp1, u, hsum, tb, wb, rels, Wr)


def kernel(toks, unberter, tokborders, wordborders, rels, emb, Wb, Wr):
    toks = toks.astype(jnp.int32)
    unberter = unberter.astype(jnp.int32)
    tokborders = tokborders.astype(jnp.int32)
    wordborders = wordborders.astype(jnp.int32)
    rels = rels.astype(jnp.int32)
    emb = emb.astype(jnp.float32)
    Wb = Wb.astype(jnp.float32)
    Wr = Wr.astype(jnp.float32)

    p0, p1 = _p_call(emb, Wb)
    hsum, bp0, bp1 = _sc_call(toks, emb, p0, p1)
    return _loss_call(
        bp0, bp1, unberter, hsum, tokborders, wordborders, rels, Wr)
